# baseline (speedup 1.0000x reference)
import jax
import jax.numpy as jnp
from jax import lax
from jax.experimental import pallas as pl
from jax.experimental.pallas import tpu as pltpu
from jax.experimental.pallas import tpu_sc as plsc

VOCAB = 100000
EMB = 64
BATCH = 4096
L = 50

NC = 2
NS = 16
NW = NC * NS

CHUNK = 128
NCHUNKS = (BATCH * L) // CHUNK
CPW = NCHUNKS // NW
NBUF = 4
CPC = BATCH // CHUNK

TV = 50176
NT = 2


def _sc_body(idx_hbm, table_hbm, out_hbm, idx_v, acc_v, *bufs_and_sems):
  bufs = bufs_and_sems[:NBUF]
  sems = bufs_and_sems[NBUF:]
  w = lax.axis_index("s") * NC + lax.axis_index("c")

  pltpu.sync_copy(idx_hbm.at[w], idx_v)

  descs = [None] * NBUF
  for k in range(NBUF):
    descs[k] = pltpu.async_copy(table_hbm.at[idx_v.at[k]], bufs[k], sems[k])

  def accumulate(buf_ref, c):
    z = jnp.zeros((16,), jnp.float32)

    def body(i, accs):
      a, b = list(accs[:4]), list(accs[4:])
      o = i * 8
      for r in range(8):
        tgt = a if (r % 2 == 0) else b
        for s in range(4):
          tgt[s] = tgt[s] + buf_ref[o + r, pl.ds(16 * s, 16)]
      return tuple(a) + tuple(b)

    accs = lax.fori_loop(0, CHUNK // 8, body, (z,) * 8)
    for s in range(4):
      acc_v[c, pl.ds(16 * s, 16)] = accs[s] + accs[4 + s]

  for c in range(CPW):
    k = c % NBUF
    descs[k].wait()
    accumulate(bufs[k], c)
    nxt = c + NBUF
    if nxt < CPW:
      descs[k] = pltpu.async_copy(table_hbm.at[idx_v.at[nxt]], bufs[k], sems[k])

  pltpu.sync_copy(acc_v, out_hbm.at[w])


def _sc_gather_sum(idx3d, table):
  mesh = plsc.VectorSubcoreMesh(core_axis_name="c", subcore_axis_name="s")
  scratch = [
      pltpu.VMEM((CPW, CHUNK), jnp.int32),
      pltpu.VMEM((CPW, EMB), jnp.float32),
  ]
  scratch += [pltpu.VMEM((CHUNK, EMB), jnp.float32) for _ in range(NBUF)]
  scratch += [pltpu.SemaphoreType.DMA for _ in range(NBUF)]
  fn = pl.kernel(
      _sc_body,
      out_type=jax.ShapeDtypeStruct((NW, CPW, EMB), jnp.float32),
      mesh=mesh,
      scratch_types=scratch,
      compiler_params=pltpu.CompilerParams(use_tc_tiling_on_sc=False),
  )
  return fn(idx3d, table)


def _tc_body(part_ref, gold_ref, w_ref, b_ref, out_ref, x_s, m_s, s_s, g_s):
  j = pl.program_id(0)

  @pl.when(j == 0)
  def _():
    p = part_ref[...].reshape(L, CPC, EMB)
    x_s[0:L, :] = jnp.sum(p, axis=1)
    x_s[L:, :] = jnp.zeros((64 - L, EMB), jnp.float32)
    m_s[...] = jnp.full((64,), -1e30, jnp.float32)
    s_s[...] = jnp.zeros((64,), jnp.float32)
    g_s[...] = jnp.zeros((64,), jnp.float32)

  x = x_s[...]
  wt = w_ref[...]
  t = lax.dot_general(
      x, wt, (((1,), (0,)), ((), ())),
      preferred_element_type=jnp.float32)
  col = j * TV + lax.broadcasted_iota(jnp.int32, (64, TV), 1)
  t = t + b_ref[...]
  t = lax.cond(j == NT - 1,
               lambda u: jnp.where(col < VOCAB, u, -1e30),
               lambda u: u, t)

  m_old = m_s[...]
  m_new = jnp.maximum(m_old, jnp.max(t, axis=1))
  p = jnp.exp(t - m_new[:, None])
  s_s[...] = s_s[...] * jnp.exp(m_old - m_new) + jnp.sum(p, axis=1)
  m_s[...] = m_new
  gmask = col == gold_ref[...][:, None]
  g_s[...] = g_s[...] + jnp.sum(jnp.where(gmask, t, 0.0), axis=1)

  @pl.when(j == NT - 1)
  def _():
    diff = m_s[...] + jnp.log(s_s[...]) - g_s[...]
    lmask = lax.broadcasted_iota(jnp.int32, (64,), 0) < L
    out_ref[0, 0] = jnp.sum(jnp.where(lmask, diff, 0.0)) / L


def _tc_dense_ce(partials, gold_pad, Wt, b2):
  return pl.pallas_call(
      _tc_body,
      grid=(NT,),
      in_specs=[
          pl.BlockSpec((NCHUNKS, EMB), lambda j: (0, 0)),
          pl.BlockSpec((64,), lambda j: (0,)),
          pl.BlockSpec((EMB, TV), lambda j: (0, j)),
          pl.BlockSpec((1, TV), lambda j: (0, j)),
      ],
      out_specs=pl.BlockSpec((1, 1), lambda j: (0, 0), memory_space=pltpu.SMEM),
      out_shape=jax.ShapeDtypeStruct((1, 1), jnp.float32),
      scratch_shapes=[
          pltpu.VMEM((64, EMB), jnp.float32),
          pltpu.VMEM((64,), jnp.float32),
          pltpu.VMEM((64,), jnp.float32),
          pltpu.VMEM((64,), jnp.float32),
      ],
  )(partials, gold_pad, Wt, b2)


def kernel(inputs, gold, emb_table, W, b):
  idx3d = inputs.T.reshape(NW, CPW, CHUNK)
  partials = _sc_gather_sum(idx3d, emb_table).reshape(NCHUNKS, EMB)
  gold_pad = jnp.concatenate([gold, jnp.zeros((64 - L,), jnp.int32)])
  loss = _tc_dense_ce(partials, gold_pad, W.T, b.reshape(1, VOCAB))
  return loss[0, 0]


# speedup vs baseline: 1.0192x; 1.0192x over previous
import jax
import jax.numpy as jnp
from jax import lax
from jax.experimental import pallas as pl
from jax.experimental.pallas import tpu as pltpu
from jax.experimental.pallas import tpu_sc as plsc

VOCAB = 100000
EMB = 64
BATCH = 4096
L = 50

NC = 2
NS = 16
NW = NC * NS

CHUNK = 128
NCHUNKS = (BATCH * L) // CHUNK
CPW = NCHUNKS // NW
NBUF = 4
CPC = BATCH // CHUNK

TV = 25088
NT = 4


def _sc_body(idx_hbm, table_hbm, out_hbm, idx_v, acc_v, *bufs_and_sems):
  bufs = bufs_and_sems[:NBUF]
  sems = bufs_and_sems[NBUF:]
  w = lax.axis_index("s") * NC + lax.axis_index("c")

  pltpu.sync_copy(idx_hbm.at[w], idx_v)

  descs = [None] * NBUF
  for k in range(NBUF):
    descs[k] = pltpu.async_copy(table_hbm.at[idx_v.at[k]], bufs[k], sems[k])

  def accumulate(buf_ref, c):
    z = jnp.zeros((16,), jnp.float32)

    def body(i, accs):
      a, b = list(accs[:4]), list(accs[4:])
      o = i * 8
      for r in range(8):
        tgt = a if (r % 2 == 0) else b
        for s in range(4):
          tgt[s] = tgt[s] + buf_ref[o + r, pl.ds(16 * s, 16)]
      return tuple(a) + tuple(b)

    accs = lax.fori_loop(0, CHUNK // 8, body, (z,) * 8)
    for s in range(4):
      acc_v[c, pl.ds(16 * s, 16)] = accs[s] + accs[4 + s]

  for c in range(CPW):
    k = c % NBUF
    descs[k].wait()
    accumulate(bufs[k], c)
    nxt = c + NBUF
    if nxt < CPW:
      descs[k] = pltpu.async_copy(table_hbm.at[idx_v.at[nxt]], bufs[k], sems[k])

  pltpu.sync_copy(acc_v, out_hbm.at[w])


def _sc_gather_sum(idx3d, table):
  mesh = plsc.VectorSubcoreMesh(core_axis_name="c", subcore_axis_name="s")
  scratch = [
      pltpu.VMEM((CPW, CHUNK), jnp.int32),
      pltpu.VMEM((CPW, EMB), jnp.float32),
  ]
  scratch += [pltpu.VMEM((CHUNK, EMB), jnp.float32) for _ in range(NBUF)]
  scratch += [pltpu.SemaphoreType.DMA for _ in range(NBUF)]
  fn = pl.kernel(
      _sc_body,
      out_type=jax.ShapeDtypeStruct((NW, CPW, EMB), jnp.float32),
      mesh=mesh,
      scratch_types=scratch,
      compiler_params=pltpu.CompilerParams(use_tc_tiling_on_sc=False),
  )
  return fn(idx3d, table)


def _tc_body(part_ref, gold_ref, w_ref, b_ref, out_ref, x_s, m_s, s_s, g_s):
  j = pl.program_id(0)

  @pl.when(j == 0)
  def _():
    p = part_ref[...].reshape(L, CPC, EMB)
    x_s[0:L, :] = jnp.sum(p, axis=1)
    x_s[L:, :] = jnp.zeros((64 - L, EMB), jnp.float32)
    m_s[...] = jnp.full((64,), -1e30, jnp.float32)
    s_s[...] = jnp.zeros((64,), jnp.float32)
    g_s[...] = jnp.zeros((64,), jnp.float32)

  x = x_s[...]
  wt = w_ref[...]
  t = lax.dot_general(
      x, wt, (((1,), (0,)), ((), ())),
      preferred_element_type=jnp.float32)
  col = j * TV + lax.broadcasted_iota(jnp.int32, (64, TV), 1)
  t = t + b_ref[...]
  t = lax.cond(j == NT - 1,
               lambda u: jnp.where(col < VOCAB, u, -1e30),
               lambda u: u, t)

  m_old = m_s[...]
  m_new = jnp.maximum(m_old, jnp.max(t, axis=1))
  p = jnp.exp(t - m_new[:, None])
  s_s[...] = s_s[...] * jnp.exp(m_old - m_new) + jnp.sum(p, axis=1)
  m_s[...] = m_new
  gmask = col == gold_ref[...][:, None]
  g_s[...] = g_s[...] + jnp.sum(jnp.where(gmask, t, 0.0), axis=1)

  @pl.when(j == NT - 1)
  def _():
    diff = m_s[...] + jnp.log(s_s[...]) - g_s[...]
    lmask = lax.broadcasted_iota(jnp.int32, (64,), 0) < L
    out_ref[0, 0] = jnp.sum(jnp.where(lmask, diff, 0.0)) / L


def _tc_dense_ce(partials, gold_pad, Wt, b2):
  return pl.pallas_call(
      _tc_body,
      grid=(NT,),
      in_specs=[
          pl.BlockSpec((NCHUNKS, EMB), lambda j: (0, 0)),
          pl.BlockSpec((64,), lambda j: (0,)),
          pl.BlockSpec((EMB, TV), lambda j: (0, j)),
          pl.BlockSpec((1, TV), lambda j: (0, j)),
      ],
      out_specs=pl.BlockSpec((1, 1), lambda j: (0, 0), memory_space=pltpu.SMEM),
      out_shape=jax.ShapeDtypeStruct((1, 1), jnp.float32),
      scratch_shapes=[
          pltpu.VMEM((64, EMB), jnp.float32),
          pltpu.VMEM((64,), jnp.float32),
          pltpu.VMEM((64,), jnp.float32),
          pltpu.VMEM((64,), jnp.float32),
      ],
  )(partials, gold_pad, Wt, b2)


def kernel(inputs, gold, emb_table, W, b):
  idx3d = inputs.T.reshape(NW, CPW, CHUNK)
  partials = _sc_gather_sum(idx3d, emb_table).reshape(NCHUNKS, EMB)
  gold_pad = jnp.concatenate([gold, jnp.zeros((64 - L,), jnp.int32)])
  loss = _tc_dense_ce(partials, gold_pad, W.T, b.reshape(1, VOCAB))
  return loss[0, 0]
